# baseline (speedup 1.0000x reference)
import functools

import numpy as np
import jax
import jax.numpy as jnp
from jax import lax
from jax.experimental import pallas as pl
from jax.experimental.pallas import tpu as pltpu
from jax.experimental.pallas import tpu_sc as plsc

_HEADING = np.pi / 2
_RAD2 = 3600
_IN2 = 3364.0
_W = 7
_D = 2 * _W + 1
_TR = 121 + 2 * _W + 1

_NC, _NS, _L = 2, 16, 16
_NW = _NC * _NS


def _build_row_tables():
    rowh = np.full((_TR,), -1, np.int64)
    pref = np.zeros((_TR,), np.int64)
    acc = 0
    for i in range(_TR):
        b = 60 - (i - _W)
        if -60 <= b <= 60:
            h = int(np.floor(np.sqrt(_RAD2 - b * b)))
            rowh[i] = h
            pref[i] = acc
            acc += 2 * h + 1
    return rowh.astype(np.int32), pref.astype(np.int32)


_ROWH, _PREF = _build_row_tables()


@functools.lru_cache(maxsize=None)
def _make_nn_kernel(q: int):
    qpw = q // _NW
    assert qpw & (qpw - 1) == 0, "qpw must be a power of two"
    nv = qpw // _L
    mesh = plsc.VectorSubcoreMesh(core_axis_name="c", subcore_axis_name="s")
    f32, i32 = jnp.float32, jnp.int32

    @functools.partial(
        pl.kernel,
        mesh=mesh,
        compiler_params=pltpu.CompilerParams(needs_layout_passes=False),
        out_type=[
            jax.ShapeDtypeStruct((q,), i32),
            jax.ShapeDtypeStruct((q,), f32),
            jax.ShapeDtypeStruct((q,), f32),
        ],
        scratch_types=[
            pltpu.VMEM((qpw,), f32),
            pltpu.VMEM((qpw,), f32),
            pltpu.VMEM((qpw,), f32),
            pltpu.VMEM((qpw,), f32),
            pltpu.VMEM((_L,), f32),
            pltpu.VMEM((_L,), f32),
            pltpu.VMEM((_TR,), i32),
            pltpu.VMEM((_TR,), i32),
            pltpu.VMEM((qpw,), f32),
            pltpu.VMEM((qpw,), f32),
            pltpu.VMEM((qpw + _L,), i32),
            pltpu.VMEM((qpw,), i32),
            pltpu.VMEM((qpw,), f32),
            pltpu.VMEM((qpw,), f32),
        ],
    )
    def nn_kernel(x0h, x1h, y0h, y1h, ch, sh, rhh, prh, oih, o0h, o1h,
                  x0v, x1v, y0v, y1v, cv_r, sv_r, rhv, prv,
                  p0v, p1v, cqv, oiv, o0v, o1v):
        wid = lax.axis_index("s") * _NC + lax.axis_index("c")
        base = wid * qpw
        pltpu.sync_copy(x0h.at[pl.ds(base, qpw)], x0v)
        pltpu.sync_copy(x1h.at[pl.ds(base, qpw)], x1v)
        pltpu.sync_copy(y0h.at[pl.ds(base, qpw)], y0v)
        pltpu.sync_copy(y1h.at[pl.ds(base, qpw)], y1v)
        pltpu.sync_copy(ch, cv_r)
        pltpu.sync_copy(sh, sv_r)
        pltpu.sync_copy(rhh, rhv)
        pltpu.sync_copy(prh, prv)
        cosv = cv_r[...]
        sinv = sv_r[...]
        lane = lax.iota(i32, _L)

        def scan_rows(p0, p1, ca, cb, dbs, w):
            d0 = [p0 - (ca + k).astype(f32) * 0.5 for k in range(-w, w + 1)]
            d0sq = [d * d for d in d0]
            bd2 = jnp.full((_L,), 1e30, f32)
            bidx = jnp.zeros((_L,), i32)
            ba = jnp.zeros((_L,), i32)
            bbst = jnp.zeros((_L,), i32)
            for db in dbs:
                bb = cb + db
                d1 = p1 - bb.astype(f32) * 0.5
                d1sq = d1 * d1
                irow = (60 + _W) - bb
                h = plsc.load_gather(rhv, [irow])
                pref = plsc.load_gather(prv, [irow])
                ph = pref + h
                nh = -h
                for k in range(-w, w + 1):
                    aa = ca + k
                    dd = d0sq[k + w] + d1sq
                    valid = (aa >= nh) & (aa <= h)
                    take = valid & (dd < bd2)
                    bd2 = jnp.where(take, dd, bd2)
                    bidx = jnp.where(take, ph + aa, bidx)
                    ba = jnp.where(take, aa, ba)
                    bbst = jnp.where(take, bb, bbst)
            return bidx, ba, bbst

        noff = jnp.int32(0)
        rot = []
        for v in range(nv):
            sl = pl.ds(v * _L, _L)
            cx = x0v[sl] - y0v[sl]
            cy = x1v[sl] - y1v[sl]
            p0 = cx * cosv - cy * sinv
            p1 = cx * sinv + cy * cosv
            p0v[sl] = p0
            p1v[sl] = p1
            rot.append((p0, p1))
            af = p0 * 2.0
            bf = p1 * 2.0
            r2 = af * af + bf * bf
            outm = r2 > _IN2
            plsc.store_compressed(cqv.at[pl.ds(noff, _L)],
                                  lane + (v * _L), mask=outm)
            noff = noff + jnp.sum(outm.astype(i32))

        for v in range(nv):
            sl = pl.ds(v * _L, _L)
            p0, p1 = rot[v]
            af = p0 * 2.0
            bf = p1 * 2.0
            ca = jnp.where(af >= 0.0, af + 0.5, af - 0.5).astype(i32)
            cb = jnp.where(bf >= 0.0, bf + 0.5, bf - 0.5).astype(i32)
            ca = jnp.clip(ca, -60, 60)
            cb = jnp.clip(cb, -60, 60)
            bidx, ba, bbst = scan_rows(p0, p1, ca, cb, (1, 0, -1), 1)
            oiv[sl] = bidx
            o0v[sl] = p0 - ba.astype(f32) * 0.5
            o1v[sl] = p1 - bbst.astype(f32) * 0.5

        def group_body(g):
            qidx = cqv[pl.ds(g * _L, _L)] & (qpw - 1)
            lmask = (lane + g * _L) < noff
            p0 = plsc.load_gather(p0v, [qidx])
            p1 = plsc.load_gather(p1v, [qidx])
            af = p0 * 2.0
            bf = p1 * 2.0
            r2 = af * af + bf * bf
            am = jnp.abs(af)
            bm = jnp.abs(bf)
            mx = jnp.maximum(am, bm)
            mn = jnp.minimum(am, bm)
            r0 = mx * 0.960434 + mn * 0.397825
            r1 = (r0 + r2 / r0) * 0.5
            scl = jnp.where(r2 <= float(_RAD2), 1.0, 60.0 / r1)
            caf = af * scl
            cbf = bf * scl
            ca = jnp.where(caf >= 0.0, caf + 0.5, caf - 0.5).astype(i32)
            cb = jnp.where(cbf >= 0.0, cbf + 0.5, cbf - 0.5).astype(i32)
            ca = jnp.clip(ca, -60, 60)
            cb = jnp.clip(cb, -60, 60)
            bidx, ba, bbst = scan_rows(p0, p1, ca, cb,
                                       tuple(range(_W, -_W - 1, -1)), _W)
            plsc.store_scatter(oiv, [qidx], bidx, mask=lmask)
            plsc.store_scatter(o0v, [qidx], p0 - ba.astype(f32) * 0.5,
                               mask=lmask)
            plsc.store_scatter(o1v, [qidx], p1 - bbst.astype(f32) * 0.5,
                               mask=lmask)

        def while_cond(g):
            return g * _L < noff

        def while_body(g):
            group_body(g)
            return g + 1

        lax.while_loop(while_cond, while_body, jnp.int32(0))

        pltpu.sync_copy(oiv, oih.at[pl.ds(base, qpw)])
        pltpu.sync_copy(o0v, o0h.at[pl.ds(base, qpw)])
        pltpu.sync_copy(o1v, o1h.at[pl.ds(base, qpw)])

    return nn_kernel


def kernel(x, y, theta_y, grid):
    del grid
    q = x.shape[0]
    th = -(theta_y.astype(jnp.float32) - _HEADING)
    cos16 = jnp.broadcast_to(jnp.cos(th), (_L,))
    sin16 = jnp.broadcast_to(jnp.sin(th), (_L,))
    x0 = x[:, 0]
    x1 = x[:, 1]
    y0 = y[:, 0]
    y1 = y[:, 1]
    rowh = jnp.asarray(_ROWH)
    pref = jnp.asarray(_PREF)
    idx, o0, o1 = _make_nn_kernel(q)(
        x0, x1, y0, y1, cos16, sin16, rowh, pref)
    return idx, jnp.stack([o0, o1], axis=-1)


# speedup vs baseline: 1.1047x; 1.1047x over previous
import functools

import numpy as np
import jax
import jax.numpy as jnp
from jax import lax
from jax.experimental import pallas as pl
from jax.experimental.pallas import tpu as pltpu
from jax.experimental.pallas import tpu_sc as plsc

_HEADING = np.pi / 2
_N = 121
_RAD2 = 3600
_IN2 = 3364.0
_W = 7
_D = 2 * _W + 1
_TP = _N + 2 * _W
_TLEN = _TP * _TP
_TPAD = (-_TLEN) % 8

_NC, _NS, _L = 2, 16, 16
_NW = _NC * _NS


def _build_tables():
    ii, jj = np.meshgrid(np.arange(_N), np.arange(_N), indexing="ij")
    a = jj - 60
    b = 60 - ii
    mask = (a * a + b * b) <= _RAD2
    cidx = np.cumsum(mask.ravel()) - 1
    tab = np.where(mask.ravel(), cidx, -1).astype(np.int32).reshape(_N, _N)
    tabp = np.full((_TP, _TP), -1, np.int32)
    tabp[_W:_W + _N, _W:_W + _N] = tab
    flat = np.concatenate(
        [tabp.ravel(), np.full((_TPAD,), -1, np.int32)])
    g = mask.ravel().sum()
    gpad = (-int(g)) % 8
    a_of = np.concatenate(
        [a.ravel()[mask.ravel()], np.zeros((gpad,), int)]).astype(np.int64)
    b_of = np.concatenate(
        [b.ravel()[mask.ravel()], np.zeros((gpad,), int)]).astype(np.int64)
    ab = ((b_of << 16) | (a_of & 0xFFFF)).astype(np.uint32).view(np.int32)
    return flat, ab


_TABLE, _AB_OF = _build_tables()
_GLEN = _AB_OF.shape[0]


@functools.lru_cache(maxsize=None)
def _make_nn_kernel(q: int):
    qpw = q // _NW
    assert qpw & (qpw - 1) == 0, "qpw must be a power of two"
    nv = qpw // _L
    mesh = plsc.VectorSubcoreMesh(core_axis_name="c", subcore_axis_name="s")
    f32, i32 = jnp.float32, jnp.int32

    @functools.partial(
        pl.kernel,
        mesh=mesh,
        compiler_params=pltpu.CompilerParams(needs_layout_passes=False),
        out_type=[
            jax.ShapeDtypeStruct((q,), i32),
            jax.ShapeDtypeStruct((q,), f32),
            jax.ShapeDtypeStruct((q,), f32),
        ],
        scratch_types=[
            pltpu.VMEM((qpw,), f32),
            pltpu.VMEM((qpw,), f32),
            pltpu.VMEM((qpw,), f32),
            pltpu.VMEM((qpw,), f32),
            pltpu.VMEM((_L,), f32),
            pltpu.VMEM((_L,), f32),
            pltpu.VMEM((_TLEN + _TPAD,), i32),
            pltpu.VMEM((_GLEN,), i32),
            pltpu.VMEM((qpw,), f32),
            pltpu.VMEM((qpw,), f32),
            pltpu.VMEM((qpw + _L,), i32),
            pltpu.VMEM((qpw,), i32),
            pltpu.VMEM((qpw,), f32),
            pltpu.VMEM((qpw,), f32),
            pltpu.SemaphoreType.DMA,
        ],
    )
    def nn_kernel(x0h, x1h, y0h, y1h, ch, sh, tabh, abh, oih, o0h, o1h,
                  x0v, x1v, y0v, y1v, cv_r, sv_r, tabv, abv,
                  p0v, p1v, cqv, oiv, o0v, o1v, tsem):
        wid = lax.axis_index("s") * _NC + lax.axis_index("c")
        base = wid * qpw
        tcp1 = pltpu.async_copy(tabh, tabv, tsem)
        tcp2 = pltpu.async_copy(abh, abv, tsem)
        pltpu.sync_copy(x0h.at[pl.ds(base, qpw)], x0v)
        pltpu.sync_copy(x1h.at[pl.ds(base, qpw)], x1v)
        pltpu.sync_copy(y0h.at[pl.ds(base, qpw)], y0v)
        pltpu.sync_copy(y1h.at[pl.ds(base, qpw)], y1v)
        pltpu.sync_copy(ch, cv_r)
        pltpu.sync_copy(sh, sv_r)
        cosv = cv_r[...]
        sinv = sv_r[...]
        lane = lax.iota(i32, _L)

        noff = jnp.int32(0)
        rot = []
        for v in range(nv):
            sl = pl.ds(v * _L, _L)
            cx = x0v[sl] - y0v[sl]
            cy = x1v[sl] - y1v[sl]
            p0 = cx * cosv - cy * sinv
            p1 = cx * sinv + cy * cosv
            p0v[sl] = p0
            p1v[sl] = p1
            rot.append((p0, p1))
            af = p0 * 2.0
            bf = p1 * 2.0
            r2 = af * af + bf * bf
            outm = r2 > _IN2
            plsc.store_compressed(cqv.at[pl.ds(noff, _L)],
                                  lane + (v * _L), mask=outm)
            noff = noff + jnp.sum(outm.astype(i32))

        tcp1.wait()
        tcp2.wait()

        for v in range(nv):
            sl = pl.ds(v * _L, _L)
            p0, p1 = rot[v]
            af = p0 * 2.0
            bf = p1 * 2.0
            ca = jnp.where(af >= 0.0, af + 0.5, af - 0.5).astype(i32)
            cb = jnp.where(bf >= 0.0, bf + 0.5, bf - 0.5).astype(i32)
            ca = jnp.clip(ca, -60, 60)
            cb = jnp.clip(cb, -60, 60)
            d0sq = []
            for k in (-1, 0, 1):
                d0_k = p0 - (ca + k).astype(f32) * 0.5
                d0sq.append(d0_k * d0_k)
            flat0_col = ca + (60 + _W)
            bd2 = jnp.full((_L,), 1e30, f32)
            bidx = jnp.zeros((_L,), i32)
            for db in (1, 0, -1):
                bb = cb + db
                d1 = p1 - bb.astype(f32) * 0.5
                d1sq = d1 * d1
                flat0 = (60 + _W - bb) * _TP + flat0_col
                for k in (-1, 0, 1):
                    gidx = plsc.load_gather(tabv, [flat0 + k])
                    dd = d0sq[k + 1] + d1sq
                    take = (gidx >= 0) & (dd < bd2)
                    bd2 = jnp.where(take, dd, bd2)
                    bidx = jnp.where(take, gidx, bidx)
            ab = plsc.load_gather(abv, [bidx])
            wb = lax.shift_right_arithmetic(ab, 16)
            wa = lax.shift_right_arithmetic(lax.shift_left(ab, 16), 16)
            oiv[sl] = bidx
            o0v[sl] = p0 - wa.astype(f32) * 0.5
            o1v[sl] = p1 - wb.astype(f32) * 0.5

        def group_body(g):
            qidx = cqv[pl.ds(g * _L, _L)] & (qpw - 1)
            lmask = (lane + g * _L) < noff
            p0 = plsc.load_gather(p0v, [qidx])
            p1 = plsc.load_gather(p1v, [qidx])
            af = p0 * 2.0
            bf = p1 * 2.0
            r2 = af * af + bf * bf
            am = jnp.abs(af)
            bm = jnp.abs(bf)
            mx = jnp.maximum(am, bm)
            mn = jnp.minimum(am, bm)
            r0 = mx * 0.960434 + mn * 0.397825
            r1 = (r0 + r2 / r0) * 0.5
            scl = jnp.where(r2 <= float(_RAD2), 1.0, 60.0 / r1)
            caf = af * scl
            cbf = bf * scl
            ca = jnp.where(caf >= 0.0, caf + 0.5, caf - 0.5).astype(i32)
            cb = jnp.where(cbf >= 0.0, cbf + 0.5, cbf - 0.5).astype(i32)
            ca = jnp.clip(ca, -60, 60)
            cb = jnp.clip(cb, -60, 60)
            d0sq = []
            for k in range(_D):
                d0_k = p0 - (ca + (k - _W)).astype(f32) * 0.5
                d0sq.append(d0_k * d0_k)
            flat0_col = ca + 60

            def row_body(r, carry):
                bd2, bidx = carry
                bb = cb + (_W - r)
                d1 = p1 - bb.astype(f32) * 0.5
                d1sq = d1 * d1
                flat0 = (60 + _W - bb) * _TP + (flat0_col + _W)
                for k in range(_D):
                    gidx = plsc.load_gather(tabv, [flat0 + (k - _W)])
                    dd = d0sq[k] + d1sq
                    take = (gidx >= 0) & (dd < bd2)
                    bd2 = jnp.where(take, dd, bd2)
                    bidx = jnp.where(take, gidx, bidx)
                return (bd2, bidx)

            init = (jnp.full((_L,), 1e30, f32), jnp.zeros((_L,), i32))
            bd2, bidx = lax.fori_loop(0, _D, row_body, init)
            ab = plsc.load_gather(abv, [bidx])
            wb = lax.shift_right_arithmetic(ab, 16)
            wa = lax.shift_right_arithmetic(lax.shift_left(ab, 16), 16)
            plsc.store_scatter(oiv, [qidx], bidx, mask=lmask)
            plsc.store_scatter(o0v, [qidx], p0 - wa.astype(f32) * 0.5,
                               mask=lmask)
            plsc.store_scatter(o1v, [qidx], p1 - wb.astype(f32) * 0.5,
                               mask=lmask)

        def while_cond(g):
            return g * _L < noff

        def while_body(g):
            group_body(g)
            return g + 1

        lax.while_loop(while_cond, while_body, jnp.int32(0))

        pltpu.sync_copy(oiv, oih.at[pl.ds(base, qpw)])
        pltpu.sync_copy(o0v, o0h.at[pl.ds(base, qpw)])
        pltpu.sync_copy(o1v, o1h.at[pl.ds(base, qpw)])

    return nn_kernel


def kernel(x, y, theta_y, grid):
    del grid
    q = x.shape[0]
    th = -(theta_y.astype(jnp.float32) - _HEADING)
    cos16 = jnp.broadcast_to(jnp.cos(th), (_L,))
    sin16 = jnp.broadcast_to(jnp.sin(th), (_L,))
    x0 = x[:, 0]
    x1 = x[:, 1]
    y0 = y[:, 0]
    y1 = y[:, 1]
    tab = jnp.asarray(_TABLE)
    ab_of = jnp.asarray(_AB_OF)
    idx, o0, o1 = _make_nn_kernel(q)(
        x0, x1, y0, y1, cos16, sin16, tab, ab_of)
    return idx, jnp.stack([o0, o1], axis=-1)
